# Initial kernel scaffold; baseline (speedup 1.0000x reference)
import jax
import jax.numpy as jnp
from jax.experimental import pallas as pl


def kernel(node_feats, edge_index, W, b):
    raise NotImplementedError("write your pallas kernel here")


# speedup vs baseline: 4.2419x; 4.2419x over previous
import functools

import jax
import jax.numpy as jnp
from jax import lax
from jax.experimental import pallas as pl
from jax.experimental.pallas import tpu as pltpu
from jax.experimental.pallas import tpu_sc as plsc

N = 10000
E = 320000
C = 128
CA = 144
NUM_CORES = 2
NUM_SUBCORES = 16
NW = NUM_CORES * NUM_SUBCORES
EW = E // NW
K = 128
NB = 80
EP = NW * NB * K
RPT = 632
N_ACC = RPT * NUM_SUBCORES


def _sc_aggregate(x_aug, row3, col3, zeros_nca):
  mesh = plsc.VectorSubcoreMesh(core_axis_name="c", subcore_axis_name="s")

  @functools.partial(
      pl.kernel,
      out_type=jax.ShapeDtypeStruct((NUM_CORES, N_ACC, CA), jnp.float32),
      mesh=mesh,
      compiler_params=pltpu.CompilerParams(use_tc_tiling_on_sc=False),
      scratch_types=[
          pltpu.VMEM((4, K), jnp.int32),
          pltpu.VMEM((4, K), jnp.int32),
          pltpu.VMEM((2, K, CA), jnp.float32),
          pltpu.VMEM_SHARED((N_ACC, CA), jnp.float32),
          pltpu.SemaphoreType.DMA,
          pltpu.SemaphoreType.DMA,
          pltpu.SemaphoreType.DMA,
      ],
  )
  def body(x_hbm, row_hbm, col_hbm, zero_hbm, out_hbm,
           row_v, col_v, gbuf, acc_sh, sem_i, sem_g, sem_s):
    cid = lax.axis_index("c")
    sid = lax.axis_index("s")
    wid = cid * NUM_SUBCORES + sid

    pltpu.sync_copy(zero_hbm.at[pl.ds(sid * RPT, RPT)],
                    acc_sh.at[pl.ds(sid * RPT, RPT)])
    pltpu.async_copy(row_hbm.at[wid, 0], row_v.at[0], sem_i)
    pltpu.async_copy(col_hbm.at[wid, 0], col_v.at[0], sem_i)
    pltpu.async_copy(row_hbm.at[wid, 1], row_v.at[1], sem_i)
    pltpu.async_copy(col_hbm.at[wid, 1], col_v.at[1], sem_i)
    plsc.subcore_barrier()
    pltpu.make_async_copy(row_hbm.at[wid, 0], row_v.at[0], sem_i).wait()
    pltpu.make_async_copy(col_hbm.at[wid, 0], col_v.at[0], sem_i).wait()
    pltpu.async_copy(x_hbm.at[col_v.at[0]], gbuf.at[0], sem_g)

    def step(j, _):
      s2 = lax.rem(j, 2)
      ns2 = lax.rem(j + 1, 2)
      s4 = lax.rem(j, 4)

      @pl.when(j >= 1)
      def _():
        pltpu.make_async_copy(
            gbuf.at[ns2], acc_sh.at[row_v.at[lax.rem(j + 3, 4)]],
            sem_s).wait()

      @pl.when(j + 1 < NB)
      def _():
        pltpu.make_async_copy(row_hbm.at[wid, j + 1],
                              row_v.at[lax.rem(j + 1, 4)], sem_i).wait()
        pltpu.make_async_copy(col_hbm.at[wid, j + 1],
                              col_v.at[lax.rem(j + 1, 4)], sem_i).wait()
        pltpu.async_copy(x_hbm.at[col_v.at[lax.rem(j + 1, 4)]],
                         gbuf.at[ns2], sem_g)

      @pl.when(j + 2 < NB)
      def _():
        pltpu.async_copy(row_hbm.at[wid, j + 2],
                         row_v.at[lax.rem(j + 2, 4)], sem_i)
        pltpu.async_copy(col_hbm.at[wid, j + 2],
                         col_v.at[lax.rem(j + 2, 4)], sem_i)

      pltpu.make_async_copy(x_hbm.at[col_v.at[s4]], gbuf.at[s2],
                            sem_g).wait()
      pltpu.async_copy(gbuf.at[s2], acc_sh.at[row_v.at[s4]], sem_s,
                       add=True)
      return 0

    lax.fori_loop(0, NB, step, 0)
    pltpu.make_async_copy(gbuf.at[(NB - 1) % 2],
                          acc_sh.at[row_v.at[(NB - 1) % 4]], sem_s).wait()
    plsc.subcore_barrier()

    pltpu.sync_copy(acc_sh.at[pl.ds(sid * RPT, RPT)],
                    out_hbm.at[cid, pl.ds(sid * RPT, RPT)])

  return body(x_aug, row3, col3, zeros_nca)


def _combine(agg0, agg1, deg0, deg1, w_t, b_row):
  BM = 1000

  def body(a0, a1, d0, d1, wt, bb, o):
    s = a0[...] + a1[...]
    d = d0[...] + d1[...]
    o[...] = jnp.dot(s / d, wt[...],
                     preferred_element_type=jnp.float32) + bb[...]

  return pl.pallas_call(
      body,
      grid=(N // BM,),
      in_specs=[
          pl.BlockSpec((BM, C), lambda i: (i, 0)),
          pl.BlockSpec((BM, C), lambda i: (i, 0)),
          pl.BlockSpec((BM, 1), lambda i: (i, 0)),
          pl.BlockSpec((BM, 1), lambda i: (i, 0)),
          pl.BlockSpec((C, C), lambda i: (0, 0)),
          pl.BlockSpec((1, C), lambda i: (0, 0)),
      ],
      out_specs=pl.BlockSpec((BM, C), lambda i: (i, 0)),
      out_shape=jax.ShapeDtypeStruct((N, C), jnp.float32),
  )(agg0, agg1, deg0, deg1, w_t, b_row)


def kernel(node_feats, edge_index, W, b):
  x_aug = jnp.zeros((N, CA), jnp.float32)
  x_aug = x_aug.at[:, :C].set(node_feats)
  x_aug = x_aug.at[:, C].set(1.0)
  pad = EP - E
  row3 = jnp.concatenate(
      [edge_index[0], jnp.full((pad,), N, jnp.int32)]).reshape(NW, NB, K)
  col3 = jnp.concatenate(
      [edge_index[1], jnp.zeros((pad,), jnp.int32)]).reshape(NW, NB, K)
  zeros_nca = jnp.zeros((N_ACC, CA), jnp.float32)

  part = _sc_aggregate(x_aug, row3, col3, zeros_nca)

  agg0 = part[0, :N, :C]
  agg1 = part[1, :N, :C]
  deg0 = part[0, :N, C:C + 1]
  deg1 = part[1, :N, C:C + 1]
  return _combine(agg0, agg1, deg0, deg1, W.T, b.reshape(1, C))
